# baseline (speedup 1.0000x reference)
import functools

import jax
import jax.numpy as jnp
from jax import lax
from jax.experimental import pallas as pl
from jax.experimental.pallas import tpu as pltpu
from jax.experimental.pallas import tpu_sc as plsc

HIDDEN = 128
BATCH = 16384

_NC = 2
_NS = 16
_NW = _NC * _NS
_BPW = BATCH // _NW
_NCH = 8
_CH = _BPW // _NCH

_mesh = plsc.VectorSubcoreMesh(core_axis_name="c", subcore_axis_name="s")


@functools.partial(
    pl.kernel,
    mesh=_mesh,
    out_type=jax.ShapeDtypeStruct((BATCH, HIDDEN), jnp.float32),
    scratch_types=[
        pltpu.VMEM((_NCH, _CH), jnp.int32),
        pltpu.VMEM_SHARED((2, HIDDEN), jnp.float32),
        pltpu.VMEM((_BPW, HIDDEN), jnp.float32),
        pltpu.SemaphoreType.DMA,
        pltpu.SemaphoreType.DMA,
        pltpu.SemaphoreType.DMA,
        pltpu.SemaphoreType.DMA,
        pltpu.SemaphoreType.DMA,
        pltpu.SemaphoreType.DMA,
        pltpu.SemaphoreType.DMA,
        pltpu.SemaphoreType.DMA,
        pltpu.SemaphoreType.DMA,
        pltpu.SemaphoreType.DMA,
        pltpu.SemaphoreType.DMA,
        pltpu.SemaphoreType.DMA,
        pltpu.SemaphoreType.DMA,
        pltpu.SemaphoreType.DMA,
        pltpu.SemaphoreType.DMA,
        pltpu.SemaphoreType.DMA,
        pltpu.SemaphoreType.DMA,
    ],
)
def _embed(labels_hbm, table_hbm, out_hbm, idx_v, tab_v, rows_v,
           lsem, *sems):
    gsem = sems[:8]
    wsem = sems[8:]
    wid = lax.axis_index("s") * _NC + lax.axis_index("c")
    base = wid * _BPW

    lcp = pltpu.async_copy(labels_hbm.at[wid], idx_v, lsem)

    @pl.when(lax.axis_index("s") == 0)
    def _():
        pltpu.sync_copy(table_hbm, tab_v)

    plsc.subcore_barrier()
    lcp.wait()

    gcp = []
    for j in range(_NCH):
        gcp.append(pltpu.async_copy(
            tab_v.at[idx_v.at[j]], rows_v.at[pl.ds(j * _CH, _CH)], gsem[j]))
    wcp = []
    for j in range(_NCH):
        gcp[j].wait()
        wcp.append(pltpu.async_copy(
            rows_v.at[pl.ds(j * _CH, _CH)],
            out_hbm.at[pl.ds(base + j * _CH, _CH)], wsem[j]))
    for j in range(_NCH):
        wcp[j].wait()


def kernel(labels, table):
    out = _embed(labels.astype(jnp.int32).reshape(_NW, _NCH, _CH), table)
    return out[:, None, :]


# speedup vs baseline: 1.0125x; 1.0125x over previous
import functools

import jax
import jax.numpy as jnp
from jax import lax
from jax.experimental import pallas as pl
from jax.experimental.pallas import tpu as pltpu
from jax.experimental.pallas import tpu_sc as plsc

HIDDEN = 128
BATCH = 16384

_NC = 2
_NS = 16
_NW = _NC * _NS
_BPW = BATCH // _NW
_NCH = 4
_CH = _BPW // _NCH

_mesh = plsc.VectorSubcoreMesh(core_axis_name="c", subcore_axis_name="s")


@functools.partial(
    pl.kernel,
    mesh=_mesh,
    out_type=jax.ShapeDtypeStruct((BATCH, HIDDEN), jnp.float32),
    scratch_types=[
        pltpu.VMEM((_NCH, _CH), jnp.int32),
        pltpu.VMEM_SHARED((2, HIDDEN), jnp.float32),
        pltpu.VMEM((_BPW, HIDDEN), jnp.float32),
        pltpu.SemaphoreType.DMA,
        pltpu.SemaphoreType.DMA,
        pltpu.SemaphoreType.DMA,
        pltpu.SemaphoreType.DMA,
        pltpu.SemaphoreType.DMA,
        pltpu.SemaphoreType.DMA,
        pltpu.SemaphoreType.DMA,
        pltpu.SemaphoreType.DMA,
        pltpu.SemaphoreType.DMA,
        pltpu.SemaphoreType.DMA,
        pltpu.SemaphoreType.DMA,
        pltpu.SemaphoreType.DMA,
    ],
)
def _embed(labels_hbm, table_hbm, out_hbm, idx_v, tab_v, rows_v, *sems):
    lsem = sems[:4]
    gsem = sems[4:8]
    wsem = sems[8:]
    wid = lax.axis_index("s") * _NC + lax.axis_index("c")
    base = wid * _BPW

    lcp = []
    for j in range(_NCH):
        lcp.append(pltpu.async_copy(
            labels_hbm.at[wid].at[j], idx_v.at[j], lsem[j]))

    @pl.when(lax.axis_index("s") == 0)
    def _():
        pltpu.sync_copy(table_hbm, tab_v)

    plsc.subcore_barrier()

    gcp = []
    for j in range(_NCH):
        lcp[j].wait()
        gcp.append(pltpu.async_copy(
            tab_v.at[idx_v.at[j]], rows_v.at[pl.ds(j * _CH, _CH)], gsem[j]))
    wcp = []
    for j in range(_NCH):
        gcp[j].wait()
        wcp.append(pltpu.async_copy(
            rows_v.at[pl.ds(j * _CH, _CH)],
            out_hbm.at[pl.ds(base + j * _CH, _CH)], wsem[j]))
    for j in range(_NCH):
        wcp[j].wait()


def kernel(labels, table):
    out = _embed(labels.astype(jnp.int32).reshape(_NW, _NCH, _CH), table)
    return out[:, None, :]
